# Initial kernel scaffold; baseline (speedup 1.0000x reference)
import jax
import jax.numpy as jnp
from jax.experimental import pallas as pl


def kernel(x, node_attr, edge_index, edge_attr_sh, edge_scalars, W_sc, W_lin1, fc_w1, fc_w2, W_lin2, W_alpha):
    raise NotImplementedError("write your pallas kernel here")


# speedup vs baseline: 2.0998x; 2.0998x over previous
import functools
import math

import jax
import jax.numpy as jnp
from jax import lax
from jax.experimental import pallas as pl
from jax.experimental.pallas import tpu as pltpu
from jax.experimental.pallas import tpu_sc as plsc

F32 = jnp.float32

N = 10000
E = 160000
D_IN = 128
D_ATTR = 4
FC_IN = 16
FC_HIDDEN = 64

NUM_TILES = 32
CHUNK = 64
IDX_WIN = 16
CHUNKS_PER_TILE = 80
NUM_WINDOWS = CHUNKS_PER_TILE // IDX_WIN
EDGES_PER_TILE = CHUNK * CHUNKS_PER_TILE
E_PAD = NUM_TILES * EDGES_PER_TILE
N_PAD = 10112
ROWS_PER_TILE = N_PAD // 16

INV_SQRT_FANIN = 1.0 / math.sqrt(float(D_IN * D_ATTR))
EDGE_W_SCALE = 1.0 / (math.sqrt(float(FC_HIDDEN)) * 2.0)
INV_SQRT_FCIN = 1.0 / math.sqrt(float(FC_IN))


def _nodes_body(x_ref, na_ref, wcat_ref, nsc_ref, nf_ref):
    x = x_ref[...]
    na = na_ref[...]
    acc = jnp.dot(x, wcat_ref[0], preferred_element_type=F32) * na[:, 0:1]
    for j in range(1, D_ATTR):
        acc += jnp.dot(x, wcat_ref[j], preferred_element_type=F32) * na[:, j : j + 1]
    acc *= INV_SQRT_FANIN
    nsc_ref[...] = acc[:, :D_IN]
    nf_ref[...] = acc[:, D_IN:]


def _nodes_call(x, node_attr, wcat):
    blk = 1000
    grid = N // blk
    return pl.pallas_call(
        _nodes_body,
        grid=(grid,),
        in_specs=[
            pl.BlockSpec((blk, D_IN), lambda i: (i, 0)),
            pl.BlockSpec((blk, D_ATTR), lambda i: (i, 0)),
            pl.BlockSpec((D_ATTR, D_IN, 2 * D_IN), lambda i: (0, 0, 0)),
        ],
        out_specs=[
            pl.BlockSpec((blk, D_IN), lambda i: (i, 0)),
            pl.BlockSpec((blk, D_IN), lambda i: (i, 0)),
        ],
        out_shape=[
            jax.ShapeDtypeStruct((N, D_IN), F32),
            jax.ShapeDtypeStruct((N, D_IN), F32),
        ],
    )(x, node_attr, wcat)


def _edges_body(es_ref, sh_ref, w1_ref, fc2_ref, w_ref):
    h = jax.nn.silu(jnp.dot(es_ref[...], w1_ref[...], preferred_element_type=F32)
                    * INV_SQRT_FCIN)
    sh = sh_ref[...]
    acc = jnp.dot(h, fc2_ref[0], preferred_element_type=F32) * sh[:, 0:1]
    for v in range(1, D_ATTR):
        acc += jnp.dot(h, fc2_ref[v], preferred_element_type=F32) * sh[:, v : v + 1]
    w_ref[...] = acc * EDGE_W_SCALE


def _edges_call(es_pad, sh_pad, fc_w1, fc2_t):
    blk = 512
    grid = E_PAD // blk
    return pl.pallas_call(
        _edges_body,
        grid=(grid,),
        in_specs=[
            pl.BlockSpec((blk, FC_IN), lambda i: (i, 0)),
            pl.BlockSpec((blk, D_ATTR), lambda i: (i, 0)),
            pl.BlockSpec((FC_IN, FC_HIDDEN), lambda i: (0, 0)),
            pl.BlockSpec((D_ATTR, FC_HIDDEN, D_IN), lambda i: (0, 0, 0)),
        ],
        out_specs=pl.BlockSpec((blk, D_IN), lambda i: (i, 0)),
        out_shape=jax.ShapeDtypeStruct((E_PAD, D_IN), F32),
    )(es_pad, sh_pad, fc_w1, fc2_t)


def _sc_call(nf, w_edge, src_r, dst_r, z128):
    mesh = plsc.VectorSubcoreMesh(core_axis_name="c", subcore_axis_name="s")

    @functools.partial(
        pl.kernel,
        mesh=mesh,
        out_type=[
            jax.ShapeDtypeStruct((2, N_PAD, D_IN), F32),
            jax.ShapeDtypeStruct((2, N_PAD, D_IN), F32),
        ],
        scratch_types=[
            pltpu.VMEM((IDX_WIN, CHUNK), jnp.int32),
            pltpu.VMEM((IDX_WIN, CHUNK), jnp.int32),
            pltpu.VMEM((CHUNK, D_IN), F32),
            pltpu.VMEM((CHUNK, D_IN), F32),
            pltpu.VMEM((CHUNK, D_IN), F32),
            pltpu.VMEM_SHARED((N_PAD, D_IN), F32),
        ],
    )
    def sc_kernel(nf_hbm, w_hbm, src_hbm, dst_hbm, z128_hbm,
                  agg_out, cnt_out,
                  src_v, dst_v, rows_v, w_v, ones_v, agg_sh):
        c = lax.axis_index("c")
        s = lax.axis_index("s")
        wid = s * 2 + c

        ones16 = jnp.ones((16,), F32)

        @pl.loop(0, CHUNK)
        def _(r):
            @pl.loop(0, D_IN, step=16)
            def _(cc):
                ones_v[r, pl.ds(cc, 16)] = ones16

        row0 = s * ROWS_PER_TILE
        pltpu.sync_copy(z128_hbm.at[pl.ds(row0, ROWS_PER_TILE)],
                        agg_sh.at[pl.ds(row0, ROWS_PER_TILE)])
        plsc.subcore_barrier()

        @pl.loop(0, NUM_WINDOWS)
        def _(wi):
            pltpu.sync_copy(src_hbm.at[wid].at[wi], src_v)
            pltpu.sync_copy(dst_hbm.at[wid].at[wi], dst_v)

            @pl.loop(0, IDX_WIN)
            def _(j):
                base = (wid * CHUNKS_PER_TILE + wi * IDX_WIN + j) * CHUNK
                pltpu.sync_copy(w_hbm.at[pl.ds(base, CHUNK)], w_v)
                pltpu.sync_copy(nf_hbm.at[src_v.at[j]], rows_v)

                @pl.loop(0, CHUNK)
                def _(r):
                    @pl.loop(0, D_IN, step=16)
                    def _(cc):
                        rows_v[r, pl.ds(cc, 16)] = (
                            rows_v[r, pl.ds(cc, 16)] * w_v[r, pl.ds(cc, 16)]
                        )

                pltpu.sync_copy(rows_v, agg_sh.at[dst_v.at[j]], add=True)

        plsc.subcore_barrier()
        pltpu.sync_copy(agg_sh.at[pl.ds(row0, ROWS_PER_TILE)],
                        agg_out.at[c].at[pl.ds(row0, ROWS_PER_TILE)])
        plsc.subcore_barrier()

        pltpu.sync_copy(z128_hbm.at[pl.ds(row0, ROWS_PER_TILE)],
                        agg_sh.at[pl.ds(row0, ROWS_PER_TILE)])
        plsc.subcore_barrier()

        @pl.loop(0, NUM_WINDOWS)
        def _(wi):
            pltpu.sync_copy(dst_hbm.at[wid].at[wi], dst_v)

            @pl.loop(0, IDX_WIN)
            def _(j):
                pltpu.sync_copy(ones_v, agg_sh.at[dst_v.at[j]], add=True)

        plsc.subcore_barrier()
        pltpu.sync_copy(agg_sh.at[pl.ds(row0, ROWS_PER_TILE)],
                        cnt_out.at[c].at[pl.ds(row0, ROWS_PER_TILE)])

    return sc_kernel(nf, w_edge, src_r, dst_r, z128)


def _combine_body(aggp_ref, cntp_ref, nsc_ref, na_ref, w2_ref, wa_ref, out_ref):
    agg = aggp_ref[0] + aggp_ref[1]
    cnt = (cntp_ref[0] + cntp_ref[1])[:, 0:1]
    nf2 = agg / jnp.maximum(cnt, 1.0)
    na = na_ref[...]
    wa = wa_ref[...]
    conv = jnp.dot(nf2, w2_ref[0], preferred_element_type=F32) * na[:, 0:1]
    alpha = jnp.sum(nf2 * wa[0:1, :], axis=1, keepdims=True) * na[:, 0:1]
    for j in range(1, D_ATTR):
        conv += jnp.dot(nf2, w2_ref[j], preferred_element_type=F32) * na[:, j : j + 1]
        alpha += jnp.sum(nf2 * wa[j : j + 1, :], axis=1, keepdims=True) * na[:, j : j + 1]
    out_ref[...] = nsc_ref[...] + (alpha * INV_SQRT_FANIN) * (conv * INV_SQRT_FANIN)


def _combine_call(aggp, cntp, nsc, node_attr, w2_t, wa2):
    blk = 1000
    grid = N // blk
    return pl.pallas_call(
        _combine_body,
        grid=(grid,),
        in_specs=[
            pl.BlockSpec((2, blk, D_IN), lambda i: (0, i, 0)),
            pl.BlockSpec((2, blk, D_IN), lambda i: (0, i, 0)),
            pl.BlockSpec((blk, D_IN), lambda i: (i, 0)),
            pl.BlockSpec((blk, D_ATTR), lambda i: (i, 0)),
            pl.BlockSpec((D_ATTR, D_IN, D_IN), lambda i: (0, 0, 0)),
            pl.BlockSpec((D_ATTR, D_IN), lambda i: (0, 0)),
        ],
        out_specs=pl.BlockSpec((blk, D_IN), lambda i: (i, 0)),
        out_shape=jax.ShapeDtypeStruct((N, D_IN), F32),
    )(aggp, cntp, nsc, node_attr, w2_t, wa2)


def kernel(x, node_attr, edge_index, edge_attr_sh, edge_scalars,
           W_sc, W_lin1, fc_w1, fc_w2, W_lin2, W_alpha):
    wcat = jnp.concatenate(
        [W_sc.transpose(1, 0, 2), W_lin1.transpose(1, 0, 2)], axis=-1)
    fc2_t = fc_w2.reshape(FC_HIDDEN, D_IN, D_ATTR).transpose(2, 0, 1)
    w2_t = W_lin2.transpose(1, 0, 2)
    wa2 = W_alpha[:, :, 0].transpose(1, 0)

    es_pad = jnp.pad(edge_scalars, ((0, E_PAD - E), (0, 0)))
    sh_pad = jnp.pad(edge_attr_sh, ((0, E_PAD - E), (0, 0)))
    src_r = jnp.pad(edge_index[0], (0, E_PAD - E)).reshape(
        NUM_TILES, NUM_WINDOWS, IDX_WIN, CHUNK)
    dst_r = jnp.pad(edge_index[1], (0, E_PAD - E),
                    constant_values=N).reshape(NUM_TILES, NUM_WINDOWS, IDX_WIN, CHUNK)
    z128 = jnp.zeros((N_PAD, D_IN), F32)

    nsc, nf = _nodes_call(x, node_attr, wcat)
    w_edge = _edges_call(es_pad, sh_pad, fc_w1, fc2_t)
    aggp, cntp = _sc_call(nf, w_edge, src_r, dst_r, z128)
    return _combine_call(aggp, cntp, nsc, node_attr, w2_t, wa2)


# speedup vs baseline: 2.4539x; 1.1686x over previous
import functools
import math

import jax
import jax.numpy as jnp
from jax import lax
from jax.experimental import pallas as pl
from jax.experimental.pallas import tpu as pltpu
from jax.experimental.pallas import tpu_sc as plsc

F32 = jnp.float32

N = 10000
E = 160000
D_IN = 128
D_ATTR = 4
FC_IN = 16
FC_HIDDEN = 64

NUM_TILES = 32
CHUNK = 64
CHUNKS_PER_TILE = 80
IDX_WIN = 40
NUM_WINDOWS = CHUNKS_PER_TILE // IDX_WIN
EDGES_PER_TILE = CHUNK * CHUNKS_PER_TILE
E_PAD = NUM_TILES * EDGES_PER_TILE
N_PAD = 10112
ROWS_PER_TILE = N_PAD // 16

INV_SQRT_FANIN = 1.0 / math.sqrt(float(D_IN * D_ATTR))
EDGE_W_SCALE = 1.0 / (math.sqrt(float(FC_HIDDEN)) * 2.0)
INV_SQRT_FCIN = 1.0 / math.sqrt(float(FC_IN))


def _nodes_body(x_ref, na_ref, wcat_ref, nsc_ref, nf_ref):
    x = x_ref[...]
    na = na_ref[...]
    acc = jnp.dot(x, wcat_ref[0], preferred_element_type=F32) * na[:, 0:1]
    for j in range(1, D_ATTR):
        acc += jnp.dot(x, wcat_ref[j], preferred_element_type=F32) * na[:, j : j + 1]
    acc *= INV_SQRT_FANIN
    nsc_ref[...] = acc[:, :D_IN]
    nf_ref[...] = acc[:, D_IN:]


def _nodes_call(x, node_attr, wcat):
    blk = 1000
    grid = N // blk
    return pl.pallas_call(
        _nodes_body,
        grid=(grid,),
        in_specs=[
            pl.BlockSpec((blk, D_IN), lambda i: (i, 0)),
            pl.BlockSpec((blk, D_ATTR), lambda i: (i, 0)),
            pl.BlockSpec((D_ATTR, D_IN, 2 * D_IN), lambda i: (0, 0, 0)),
        ],
        out_specs=[
            pl.BlockSpec((blk, D_IN), lambda i: (i, 0)),
            pl.BlockSpec((blk, D_IN), lambda i: (i, 0)),
        ],
        out_shape=[
            jax.ShapeDtypeStruct((N, D_IN), F32),
            jax.ShapeDtypeStruct((N, D_IN), F32),
        ],
    )(x, node_attr, wcat)


def _edges_body(es_ref, sh_ref, w1_ref, fc2_ref, w_ref):
    h = jax.nn.silu(jnp.dot(es_ref[...], w1_ref[...], preferred_element_type=F32)
                    * INV_SQRT_FCIN)
    sh = sh_ref[...]
    acc = jnp.dot(h, fc2_ref[0], preferred_element_type=F32) * sh[:, 0:1]
    for v in range(1, D_ATTR):
        acc += jnp.dot(h, fc2_ref[v], preferred_element_type=F32) * sh[:, v : v + 1]
    w_ref[...] = acc * EDGE_W_SCALE


def _edges_call(es_pad, sh_pad, fc_w1, fc2_t):
    blk = 512
    grid = E_PAD // blk
    return pl.pallas_call(
        _edges_body,
        grid=(grid,),
        in_specs=[
            pl.BlockSpec((blk, FC_IN), lambda i: (i, 0)),
            pl.BlockSpec((blk, D_ATTR), lambda i: (i, 0)),
            pl.BlockSpec((FC_IN, FC_HIDDEN), lambda i: (0, 0)),
            pl.BlockSpec((D_ATTR, FC_HIDDEN, D_IN), lambda i: (0, 0, 0)),
        ],
        out_specs=pl.BlockSpec((blk, D_IN), lambda i: (i, 0)),
        out_shape=jax.ShapeDtypeStruct((E_PAD, D_IN), F32),
    )(es_pad, sh_pad, fc_w1, fc2_t)


def _sc_call(nf, w_edge, src_r, dst_r, z128):
    mesh = plsc.VectorSubcoreMesh(core_axis_name="c", subcore_axis_name="s")

    @functools.partial(
        pl.kernel,
        mesh=mesh,
        out_type=[
            jax.ShapeDtypeStruct((2, N_PAD, D_IN), F32),
            jax.ShapeDtypeStruct((2, N_PAD, D_IN), F32),
        ],
        scratch_types=[
            pltpu.VMEM((IDX_WIN, CHUNK), jnp.int32),
            pltpu.VMEM((IDX_WIN, CHUNK), jnp.int32),
            pltpu.VMEM((CHUNK, D_IN), F32),
            pltpu.VMEM((CHUNK, D_IN), F32),
            pltpu.VMEM((CHUNK, D_IN), F32),
            pltpu.VMEM((CHUNK, D_IN), F32),
            pltpu.SemaphoreType.DMA,
            pltpu.SemaphoreType.DMA,
            pltpu.SemaphoreType.DMA,
            pltpu.SemaphoreType.DMA,
            pltpu.VMEM_SHARED((N_PAD, D_IN), F32),
        ],
    )
    def sc_kernel(nf_hbm, w_hbm, src_hbm, dst_hbm, z128_hbm,
                  agg_out, cnt_out,
                  src_v, dst_v, w_v0, w_v1, rows_v0, rows_v1,
                  wsem0, wsem1, gsem0, gsem1, agg_sh):
        c = lax.axis_index("c")
        s = lax.axis_index("s")
        wid = s * 2 + c
        w_bufs = (w_v0, w_v1)
        r_bufs = (rows_v0, rows_v1)
        w_sems = (wsem0, wsem1)
        g_sems = (gsem0, gsem1)

        row0 = s * ROWS_PER_TILE

        pltpu.sync_copy(z128_hbm.at[pl.ds(row0, ROWS_PER_TILE)],
                        agg_sh.at[pl.ds(row0, ROWS_PER_TILE)])
        plsc.subcore_barrier()

        def fetch(win_base, jj, b):
            pltpu.async_copy(w_hbm.at[pl.ds(win_base + jj * CHUNK, CHUNK)],
                             w_bufs[b], w_sems[b])
            pltpu.async_copy(nf_hbm.at[src_v.at[jj]], r_bufs[b], g_sems[b])

        def consume(win_base, jj, b):
            pltpu.make_async_copy(w_hbm.at[pl.ds(win_base, CHUNK)],
                                  w_bufs[b], w_sems[b]).wait()
            pltpu.make_async_copy(nf_hbm.at[src_v.at[jj]],
                                  r_bufs[b], g_sems[b]).wait()
            rv, wv = r_bufs[b], w_bufs[b]

            @pl.loop(0, CHUNK)
            def _(r):
                @pl.loop(0, D_IN, step=16)
                def _(cc):
                    rv[r, pl.ds(cc, 16)] = (
                        rv[r, pl.ds(cc, 16)] * wv[r, pl.ds(cc, 16)]
                    )

            pltpu.sync_copy(rv, agg_sh.at[dst_v.at[jj]], add=True)

        for wi in range(NUM_WINDOWS):
            win_base = (wid * CHUNKS_PER_TILE + wi * IDX_WIN) * CHUNK
            pltpu.sync_copy(src_hbm.at[wid].at[wi], src_v)
            pltpu.sync_copy(dst_hbm.at[wid].at[wi], dst_v)

            for b in range(2):
                fetch(win_base, b, b)

            @pl.loop(0, IDX_WIN - 2, step=2)
            def _(j):
                for b in range(2):
                    consume(win_base, j + b, b)
                    fetch(win_base, j + b + 2, b)

            for b in range(2):
                consume(win_base, IDX_WIN - 2 + b, b)

        plsc.subcore_barrier()
        pltpu.sync_copy(agg_sh.at[pl.ds(row0, ROWS_PER_TILE)],
                        agg_out.at[c].at[pl.ds(row0, ROWS_PER_TILE)])
        plsc.subcore_barrier()

        ones16 = jnp.ones((16,), F32)

        @pl.loop(0, CHUNK)
        def _(r):
            @pl.loop(0, D_IN, step=16)
            def _(cc):
                rows_v0[r, pl.ds(cc, 16)] = ones16

        pltpu.sync_copy(z128_hbm.at[pl.ds(row0, ROWS_PER_TILE)],
                        agg_sh.at[pl.ds(row0, ROWS_PER_TILE)])
        plsc.subcore_barrier()

        for wi in range(NUM_WINDOWS):
            pltpu.sync_copy(dst_hbm.at[wid].at[wi], dst_v)

            @pl.loop(0, IDX_WIN)
            def _(j):
                pltpu.sync_copy(rows_v0, agg_sh.at[dst_v.at[j]], add=True)

        plsc.subcore_barrier()
        pltpu.sync_copy(agg_sh.at[pl.ds(row0, ROWS_PER_TILE)],
                        cnt_out.at[c].at[pl.ds(row0, ROWS_PER_TILE)])

    return sc_kernel(nf, w_edge, src_r, dst_r, z128)


def _combine_body(aggp_ref, cntp_ref, nsc_ref, na_ref, w2_ref, wa_ref, out_ref):
    agg = aggp_ref[0] + aggp_ref[1]
    cnt = (cntp_ref[0] + cntp_ref[1])[:, 0:1]
    nf2 = agg / jnp.maximum(cnt, 1.0)
    na = na_ref[...]
    wa = wa_ref[...]
    conv = jnp.dot(nf2, w2_ref[0], preferred_element_type=F32) * na[:, 0:1]
    alpha = jnp.sum(nf2 * wa[0:1, :], axis=1, keepdims=True) * na[:, 0:1]
    for j in range(1, D_ATTR):
        conv += jnp.dot(nf2, w2_ref[j], preferred_element_type=F32) * na[:, j : j + 1]
        alpha += jnp.sum(nf2 * wa[j : j + 1, :], axis=1, keepdims=True) * na[:, j : j + 1]
    out_ref[...] = nsc_ref[...] + (alpha * INV_SQRT_FANIN) * (conv * INV_SQRT_FANIN)


def _combine_call(aggp, cntp, nsc, node_attr, w2_t, wa2):
    blk = 1000
    grid = N // blk
    return pl.pallas_call(
        _combine_body,
        grid=(grid,),
        in_specs=[
            pl.BlockSpec((2, blk, D_IN), lambda i: (0, i, 0)),
            pl.BlockSpec((2, blk, D_IN), lambda i: (0, i, 0)),
            pl.BlockSpec((blk, D_IN), lambda i: (i, 0)),
            pl.BlockSpec((blk, D_ATTR), lambda i: (i, 0)),
            pl.BlockSpec((D_ATTR, D_IN, D_IN), lambda i: (0, 0, 0)),
            pl.BlockSpec((D_ATTR, D_IN), lambda i: (0, 0)),
        ],
        out_specs=pl.BlockSpec((blk, D_IN), lambda i: (i, 0)),
        out_shape=jax.ShapeDtypeStruct((N, D_IN), F32),
    )(aggp, cntp, nsc, node_attr, w2_t, wa2)


def kernel(x, node_attr, edge_index, edge_attr_sh, edge_scalars,
           W_sc, W_lin1, fc_w1, fc_w2, W_lin2, W_alpha):
    wcat = jnp.concatenate(
        [W_sc.transpose(1, 0, 2), W_lin1.transpose(1, 0, 2)], axis=-1)
    fc2_t = fc_w2.reshape(FC_HIDDEN, D_IN, D_ATTR).transpose(2, 0, 1)
    w2_t = W_lin2.transpose(1, 0, 2)
    wa2 = W_alpha[:, :, 0].transpose(1, 0)

    es_pad = jnp.pad(edge_scalars, ((0, E_PAD - E), (0, 0)))
    sh_pad = jnp.pad(edge_attr_sh, ((0, E_PAD - E), (0, 0)))
    src_r = jnp.pad(edge_index[0], (0, E_PAD - E)).reshape(
        NUM_TILES, NUM_WINDOWS, IDX_WIN, CHUNK)
    dst_r = jnp.pad(edge_index[1], (0, E_PAD - E),
                    constant_values=N).reshape(NUM_TILES, NUM_WINDOWS, IDX_WIN, CHUNK)
    z128 = jnp.zeros((N_PAD, D_IN), F32)

    nsc, nf = _nodes_call(x, node_attr, wcat)
    w_edge = _edges_call(es_pad, sh_pad, fc_w1, fc2_t)
    aggp, cntp = _sc_call(nf, w_edge, src_r, dst_r, z128)
    return _combine_call(aggp, cntp, nsc, node_attr, w2_t, wa2)


# speedup vs baseline: 2.6022x; 1.0604x over previous
import functools
import math

import jax
import jax.numpy as jnp
from jax import lax
from jax.experimental import pallas as pl
from jax.experimental.pallas import tpu as pltpu
from jax.experimental.pallas import tpu_sc as plsc

F32 = jnp.float32

N = 10000
E = 160000
D_IN = 128
D_ATTR = 4
FC_IN = 16
FC_HIDDEN = 64

NUM_TILES = 32
CHUNK = 64
CHUNKS_PER_TILE = 80
IDX_WIN = 40
NUM_WINDOWS = CHUNKS_PER_TILE // IDX_WIN
EDGES_PER_TILE = CHUNK * CHUNKS_PER_TILE
E_PAD = NUM_TILES * EDGES_PER_TILE
N_PAD = 10112
ROWS_PER_TILE = N_PAD // 16

INV_SQRT_FANIN = 1.0 / math.sqrt(float(D_IN * D_ATTR))
EDGE_W_SCALE = 1.0 / (math.sqrt(float(FC_HIDDEN)) * 2.0)
INV_SQRT_FCIN = 1.0 / math.sqrt(float(FC_IN))


def _nodes_body(x_ref, na_ref, wcat_ref, nsc_ref, nf_ref):
    x = x_ref[...]
    na = na_ref[...]
    acc = jnp.dot(x, wcat_ref[0], preferred_element_type=F32) * na[:, 0:1]
    for j in range(1, D_ATTR):
        acc += jnp.dot(x, wcat_ref[j], preferred_element_type=F32) * na[:, j : j + 1]
    acc *= INV_SQRT_FANIN
    nsc_ref[...] = acc[:, :D_IN]
    nf_ref[...] = acc[:, D_IN:]


def _nodes_call(x, node_attr, wcat):
    blk = 1000
    grid = N // blk
    return pl.pallas_call(
        _nodes_body,
        grid=(grid,),
        in_specs=[
            pl.BlockSpec((blk, D_IN), lambda i: (i, 0)),
            pl.BlockSpec((blk, D_ATTR), lambda i: (i, 0)),
            pl.BlockSpec((D_ATTR, D_IN, 2 * D_IN), lambda i: (0, 0, 0)),
        ],
        out_specs=[
            pl.BlockSpec((blk, D_IN), lambda i: (i, 0)),
            pl.BlockSpec((blk, D_IN), lambda i: (i, 0)),
        ],
        out_shape=[
            jax.ShapeDtypeStruct((N, D_IN), F32),
            jax.ShapeDtypeStruct((N, D_IN), F32),
        ],
    )(x, node_attr, wcat)


def _edges_body(es_ref, sh_ref, w1_ref, fc2_ref, w_ref):
    h = jax.nn.silu(jnp.dot(es_ref[...], w1_ref[...], preferred_element_type=F32)
                    * INV_SQRT_FCIN)
    sh = sh_ref[...]
    acc = jnp.dot(h, fc2_ref[0], preferred_element_type=F32) * sh[:, 0:1]
    for v in range(1, D_ATTR):
        acc += jnp.dot(h, fc2_ref[v], preferred_element_type=F32) * sh[:, v : v + 1]
    w_ref[...] = acc * EDGE_W_SCALE


def _edges_call(es_pad, sh_pad, fc_w1, fc2_t):
    blk = 512
    grid = E_PAD // blk
    return pl.pallas_call(
        _edges_body,
        grid=(grid,),
        in_specs=[
            pl.BlockSpec((blk, FC_IN), lambda i: (i, 0)),
            pl.BlockSpec((blk, D_ATTR), lambda i: (i, 0)),
            pl.BlockSpec((FC_IN, FC_HIDDEN), lambda i: (0, 0)),
            pl.BlockSpec((D_ATTR, FC_HIDDEN, D_IN), lambda i: (0, 0, 0)),
        ],
        out_specs=pl.BlockSpec((blk, D_IN), lambda i: (i, 0)),
        out_shape=jax.ShapeDtypeStruct((E_PAD, D_IN), F32),
    )(es_pad, sh_pad, fc_w1, fc2_t)


def _sc_cnt_call(dst_r, z128):
    mesh = plsc.VectorSubcoreMesh(core_axis_name="c", subcore_axis_name="s")

    @functools.partial(
        pl.kernel,
        mesh=mesh,
        out_type=jax.ShapeDtypeStruct((2, N_PAD, D_IN), F32),
        scratch_types=[
            pltpu.VMEM((IDX_WIN, CHUNK), jnp.int32),
            pltpu.VMEM((CHUNK, D_IN), F32),
            pltpu.VMEM_SHARED((N_PAD, D_IN), F32),
        ],
    )
    def cnt_kernel(dst_hbm, z128_hbm, cnt_out, dst_v, ones_v, agg_sh):
        c = lax.axis_index("c")
        s = lax.axis_index("s")
        wid = s * 2 + c
        row0 = s * ROWS_PER_TILE

        ones16 = jnp.ones((16,), F32)

        @pl.loop(0, CHUNK)
        def _(r):
            @pl.loop(0, D_IN, step=16)
            def _(cc):
                ones_v[r, pl.ds(cc, 16)] = ones16

        pltpu.sync_copy(z128_hbm.at[pl.ds(row0, ROWS_PER_TILE)],
                        agg_sh.at[pl.ds(row0, ROWS_PER_TILE)])
        plsc.subcore_barrier()

        for wi in range(NUM_WINDOWS):
            pltpu.sync_copy(dst_hbm.at[wid].at[wi], dst_v)

            @pl.loop(0, IDX_WIN)
            def _(j):
                pltpu.sync_copy(ones_v, agg_sh.at[dst_v.at[j]], add=True)

        plsc.subcore_barrier()
        pltpu.sync_copy(agg_sh.at[pl.ds(row0, ROWS_PER_TILE)],
                        cnt_out.at[c].at[pl.ds(row0, ROWS_PER_TILE)])

    return cnt_kernel(dst_r, z128)


def _sc_call(nf, w_edge, src_r, dst_r, z128):
    mesh = plsc.VectorSubcoreMesh(core_axis_name="c", subcore_axis_name="s")

    @functools.partial(
        pl.kernel,
        mesh=mesh,
        out_type=jax.ShapeDtypeStruct((2, N_PAD, D_IN), F32),
        scratch_types=[
            pltpu.VMEM((IDX_WIN, CHUNK), jnp.int32),
            pltpu.VMEM((IDX_WIN, CHUNK), jnp.int32),
            pltpu.VMEM((CHUNK, D_IN), F32),
            pltpu.VMEM((CHUNK, D_IN), F32),
            pltpu.VMEM((CHUNK, D_IN), F32),
            pltpu.VMEM((CHUNK, D_IN), F32),
            pltpu.SemaphoreType.DMA,
            pltpu.SemaphoreType.DMA,
            pltpu.SemaphoreType.DMA,
            pltpu.SemaphoreType.DMA,
            pltpu.VMEM_SHARED((N_PAD, D_IN), F32),
        ],
    )
    def sc_kernel(nf_hbm, w_hbm, src_hbm, dst_hbm, z128_hbm,
                  agg_out,
                  src_v, dst_v, w_v0, w_v1, rows_v0, rows_v1,
                  wsem0, wsem1, gsem0, gsem1, agg_sh):
        c = lax.axis_index("c")
        s = lax.axis_index("s")
        wid = s * 2 + c
        w_bufs = (w_v0, w_v1)
        r_bufs = (rows_v0, rows_v1)
        w_sems = (wsem0, wsem1)
        g_sems = (gsem0, gsem1)

        row0 = s * ROWS_PER_TILE

        pltpu.sync_copy(z128_hbm.at[pl.ds(row0, ROWS_PER_TILE)],
                        agg_sh.at[pl.ds(row0, ROWS_PER_TILE)])
        plsc.subcore_barrier()

        def fetch(win_base, jj, b):
            pltpu.async_copy(w_hbm.at[pl.ds(win_base + jj * CHUNK, CHUNK)],
                             w_bufs[b], w_sems[b])
            pltpu.async_copy(nf_hbm.at[src_v.at[jj]], r_bufs[b], g_sems[b])

        def consume(win_base, jj, b):
            pltpu.make_async_copy(w_hbm.at[pl.ds(win_base, CHUNK)],
                                  w_bufs[b], w_sems[b]).wait()
            pltpu.make_async_copy(nf_hbm.at[src_v.at[jj]],
                                  r_bufs[b], g_sems[b]).wait()
            rv, wv = r_bufs[b], w_bufs[b]

            @pl.loop(0, CHUNK)
            def _(r):
                @pl.loop(0, D_IN, step=16)
                def _(cc):
                    rv[r, pl.ds(cc, 16)] = (
                        rv[r, pl.ds(cc, 16)] * wv[r, pl.ds(cc, 16)]
                    )

            pltpu.sync_copy(rv, agg_sh.at[dst_v.at[jj]], add=True)

        for wi in range(NUM_WINDOWS):
            win_base = (wid * CHUNKS_PER_TILE + wi * IDX_WIN) * CHUNK
            pltpu.sync_copy(src_hbm.at[wid].at[wi], src_v)
            pltpu.sync_copy(dst_hbm.at[wid].at[wi], dst_v)

            for b in range(2):
                fetch(win_base, b, b)

            @pl.loop(0, IDX_WIN - 2, step=2)
            def _(j):
                for b in range(2):
                    consume(win_base, j + b, b)
                    fetch(win_base, j + b + 2, b)

            for b in range(2):
                consume(win_base, IDX_WIN - 2 + b, b)

        plsc.subcore_barrier()
        pltpu.sync_copy(agg_sh.at[pl.ds(row0, ROWS_PER_TILE)],
                        agg_out.at[c].at[pl.ds(row0, ROWS_PER_TILE)])

    return sc_kernel(nf, w_edge, src_r, dst_r, z128)


def _combine_body(aggp_ref, cntp_ref, nsc_ref, na_ref, w2_ref, wa_ref, out_ref):
    agg = aggp_ref[0] + aggp_ref[1]
    cnt = (cntp_ref[0] + cntp_ref[1])[:, 0:1]
    nf2 = agg / jnp.maximum(cnt, 1.0)
    na = na_ref[...]
    wa = wa_ref[...]
    conv = jnp.dot(nf2, w2_ref[0], preferred_element_type=F32) * na[:, 0:1]
    alpha = jnp.sum(nf2 * wa[0:1, :], axis=1, keepdims=True) * na[:, 0:1]
    for j in range(1, D_ATTR):
        conv += jnp.dot(nf2, w2_ref[j], preferred_element_type=F32) * na[:, j : j + 1]
        alpha += jnp.sum(nf2 * wa[j : j + 1, :], axis=1, keepdims=True) * na[:, j : j + 1]
    out_ref[...] = nsc_ref[...] + (alpha * INV_SQRT_FANIN) * (conv * INV_SQRT_FANIN)


def _combine_call(aggp, cntp, nsc, node_attr, w2_t, wa2):
    blk = 1000
    grid = N // blk
    return pl.pallas_call(
        _combine_body,
        grid=(grid,),
        in_specs=[
            pl.BlockSpec((2, blk, D_IN), lambda i: (0, i, 0)),
            pl.BlockSpec((2, blk, D_IN), lambda i: (0, i, 0)),
            pl.BlockSpec((blk, D_IN), lambda i: (i, 0)),
            pl.BlockSpec((blk, D_ATTR), lambda i: (i, 0)),
            pl.BlockSpec((D_ATTR, D_IN, D_IN), lambda i: (0, 0, 0)),
            pl.BlockSpec((D_ATTR, D_IN), lambda i: (0, 0)),
        ],
        out_specs=pl.BlockSpec((blk, D_IN), lambda i: (i, 0)),
        out_shape=jax.ShapeDtypeStruct((N, D_IN), F32),
    )(aggp, cntp, nsc, node_attr, w2_t, wa2)


def kernel(x, node_attr, edge_index, edge_attr_sh, edge_scalars,
           W_sc, W_lin1, fc_w1, fc_w2, W_lin2, W_alpha):
    wcat = jnp.concatenate(
        [W_sc.transpose(1, 0, 2), W_lin1.transpose(1, 0, 2)], axis=-1)
    fc2_t = fc_w2.reshape(FC_HIDDEN, D_IN, D_ATTR).transpose(2, 0, 1)
    w2_t = W_lin2.transpose(1, 0, 2)
    wa2 = W_alpha[:, :, 0].transpose(1, 0)

    es_pad = jnp.pad(edge_scalars, ((0, E_PAD - E), (0, 0)))
    sh_pad = jnp.pad(edge_attr_sh, ((0, E_PAD - E), (0, 0)))
    src_r = jnp.pad(edge_index[0], (0, E_PAD - E)).reshape(
        NUM_TILES, NUM_WINDOWS, IDX_WIN, CHUNK)
    dst_r = jnp.pad(edge_index[1], (0, E_PAD - E),
                    constant_values=N).reshape(NUM_TILES, NUM_WINDOWS, IDX_WIN, CHUNK)
    z128 = jnp.zeros((N_PAD, D_IN), F32)

    cntp = _sc_cnt_call(dst_r, z128)
    nsc, nf = _nodes_call(x, node_attr, wcat)
    w_edge = _edges_call(es_pad, sh_pad, fc_w1, fc2_t)
    aggp = _sc_call(nf, w_edge, src_r, dst_r, z128)
    return _combine_call(aggp, cntp, nsc, node_attr, w2_t, wa2)


# speedup vs baseline: 2.6141x; 1.0046x over previous
import functools
import math

import jax
import jax.numpy as jnp
from jax import lax
from jax.experimental import pallas as pl
from jax.experimental.pallas import tpu as pltpu
from jax.experimental.pallas import tpu_sc as plsc

F32 = jnp.float32

N = 10000
E = 160000
D_IN = 128
D_ATTR = 4
FC_IN = 16
FC_HIDDEN = 64

NUM_TILES = 32
CHUNK = 64
CHUNKS_PER_TILE = 80
IDX_WIN = 40
NUM_WINDOWS = CHUNKS_PER_TILE // IDX_WIN
EDGES_PER_TILE = CHUNK * CHUNKS_PER_TILE
E_PAD = NUM_TILES * EDGES_PER_TILE
N_PAD = 10112
ROWS_PER_TILE = N_PAD // 16

INV_SQRT_FANIN = 1.0 / math.sqrt(float(D_IN * D_ATTR))
EDGE_W_SCALE = 1.0 / (math.sqrt(float(FC_HIDDEN)) * 2.0)
INV_SQRT_FCIN = 1.0 / math.sqrt(float(FC_IN))


def _nodes_body(x_ref, na_ref, wcat_ref, nsc_ref, nf_ref):
    x = x_ref[...]
    na = na_ref[...]
    acc = jnp.dot(x, wcat_ref[0], preferred_element_type=F32) * na[:, 0:1]
    for j in range(1, D_ATTR):
        acc += jnp.dot(x, wcat_ref[j], preferred_element_type=F32) * na[:, j : j + 1]
    acc *= INV_SQRT_FANIN
    nsc_ref[...] = acc[:, :D_IN]
    nf_ref[...] = acc[:, D_IN:]


def _nodes_call(x, node_attr, wcat):
    blk = 1000
    grid = N // blk
    return pl.pallas_call(
        _nodes_body,
        grid=(grid,),
        in_specs=[
            pl.BlockSpec((blk, D_IN), lambda i: (i, 0)),
            pl.BlockSpec((blk, D_ATTR), lambda i: (i, 0)),
            pl.BlockSpec((D_ATTR, D_IN, 2 * D_IN), lambda i: (0, 0, 0)),
        ],
        out_specs=[
            pl.BlockSpec((blk, D_IN), lambda i: (i, 0)),
            pl.BlockSpec((blk, D_IN), lambda i: (i, 0)),
        ],
        out_shape=[
            jax.ShapeDtypeStruct((N, D_IN), F32),
            jax.ShapeDtypeStruct((N, D_IN), F32),
        ],
    )(x, node_attr, wcat)


def _edges_body(es_ref, sh_ref, w1_ref, fc2_ref, w_ref):
    h = jax.nn.silu(jnp.dot(es_ref[...], w1_ref[...], preferred_element_type=F32)
                    * INV_SQRT_FCIN)
    sh = sh_ref[...]
    acc = jnp.dot(h, fc2_ref[0], preferred_element_type=F32) * sh[:, 0:1]
    for v in range(1, D_ATTR):
        acc += jnp.dot(h, fc2_ref[v], preferred_element_type=F32) * sh[:, v : v + 1]
    w_ref[...] = acc * EDGE_W_SCALE


def _edges_call(es_pad, sh_pad, fc_w1, fc2_t):
    blk = 512
    grid = E_PAD // blk
    return pl.pallas_call(
        _edges_body,
        grid=(grid,),
        in_specs=[
            pl.BlockSpec((blk, FC_IN), lambda i: (i, 0)),
            pl.BlockSpec((blk, D_ATTR), lambda i: (i, 0)),
            pl.BlockSpec((FC_IN, FC_HIDDEN), lambda i: (0, 0)),
            pl.BlockSpec((D_ATTR, FC_HIDDEN, D_IN), lambda i: (0, 0, 0)),
        ],
        out_specs=pl.BlockSpec((blk, D_IN), lambda i: (i, 0)),
        out_shape=jax.ShapeDtypeStruct((E_PAD, D_IN), F32),
    )(es_pad, sh_pad, fc_w1, fc2_t)


def _sc_cnt_call(dst_r, z128):
    mesh = plsc.VectorSubcoreMesh(core_axis_name="c", subcore_axis_name="s")

    @functools.partial(
        pl.kernel,
        mesh=mesh,
        out_type=jax.ShapeDtypeStruct((2, N_PAD, D_IN), F32),
        scratch_types=[
            pltpu.VMEM((IDX_WIN, CHUNK), jnp.int32),
            pltpu.VMEM((CHUNK, D_IN), F32),
            pltpu.VMEM_SHARED((N_PAD, D_IN), F32),
        ],
    )
    def cnt_kernel(dst_hbm, z128_hbm, cnt_out, dst_v, ones_v, agg_sh):
        c = lax.axis_index("c")
        s = lax.axis_index("s")
        wid = s * 2 + c
        row0 = s * ROWS_PER_TILE

        ones16 = jnp.ones((16,), F32)

        @pl.loop(0, CHUNK)
        def _(r):
            for cc in range(0, D_IN, 16):
                ones_v[r, pl.ds(cc, 16)] = ones16

        pltpu.sync_copy(z128_hbm.at[pl.ds(row0, ROWS_PER_TILE)],
                        agg_sh.at[pl.ds(row0, ROWS_PER_TILE)])
        plsc.subcore_barrier()

        for wi in range(NUM_WINDOWS):
            pltpu.sync_copy(dst_hbm.at[wid].at[wi], dst_v)

            @pl.loop(0, IDX_WIN)
            def _(j):
                pltpu.sync_copy(ones_v, agg_sh.at[dst_v.at[j]], add=True)

        plsc.subcore_barrier()
        pltpu.sync_copy(agg_sh.at[pl.ds(row0, ROWS_PER_TILE)],
                        cnt_out.at[c].at[pl.ds(row0, ROWS_PER_TILE)])

    return cnt_kernel(dst_r, z128)


def _sc_call(nf, w_edge, src_r, dst_r, z128):
    mesh = plsc.VectorSubcoreMesh(core_axis_name="c", subcore_axis_name="s")

    @functools.partial(
        pl.kernel,
        mesh=mesh,
        out_type=jax.ShapeDtypeStruct((2, N_PAD, D_IN), F32),
        scratch_types=[
            pltpu.VMEM((IDX_WIN, CHUNK), jnp.int32),
            pltpu.VMEM((IDX_WIN, CHUNK), jnp.int32),
            pltpu.VMEM((CHUNK, D_IN), F32),
            pltpu.VMEM((CHUNK, D_IN), F32),
            pltpu.VMEM((CHUNK, D_IN), F32),
            pltpu.VMEM((CHUNK, D_IN), F32),
            pltpu.SemaphoreType.DMA,
            pltpu.SemaphoreType.DMA,
            pltpu.SemaphoreType.DMA,
            pltpu.SemaphoreType.DMA,
            pltpu.VMEM_SHARED((N_PAD, D_IN), F32),
        ],
    )
    def sc_kernel(nf_hbm, w_hbm, src_hbm, dst_hbm, z128_hbm,
                  agg_out,
                  src_v, dst_v, w_v0, w_v1, rows_v0, rows_v1,
                  wsem0, wsem1, gsem0, gsem1, agg_sh):
        c = lax.axis_index("c")
        s = lax.axis_index("s")
        wid = s * 2 + c
        w_bufs = (w_v0, w_v1)
        r_bufs = (rows_v0, rows_v1)
        w_sems = (wsem0, wsem1)
        g_sems = (gsem0, gsem1)

        row0 = s * ROWS_PER_TILE

        pltpu.sync_copy(z128_hbm.at[pl.ds(row0, ROWS_PER_TILE)],
                        agg_sh.at[pl.ds(row0, ROWS_PER_TILE)])
        plsc.subcore_barrier()

        def fetch(win_base, jj, b):
            pltpu.async_copy(w_hbm.at[pl.ds(win_base + jj * CHUNK, CHUNK)],
                             w_bufs[b], w_sems[b])
            pltpu.async_copy(nf_hbm.at[src_v.at[jj]], r_bufs[b], g_sems[b])

        def consume(win_base, jj, b):
            pltpu.make_async_copy(w_hbm.at[pl.ds(win_base, CHUNK)],
                                  w_bufs[b], w_sems[b]).wait()
            pltpu.make_async_copy(nf_hbm.at[src_v.at[jj]],
                                  r_bufs[b], g_sems[b]).wait()
            rv, wv = r_bufs[b], w_bufs[b]

            @pl.loop(0, CHUNK)
            def _(r):
                for cc in range(0, D_IN, 16):
                    rv[r, pl.ds(cc, 16)] = (
                        rv[r, pl.ds(cc, 16)] * wv[r, pl.ds(cc, 16)]
                    )

            pltpu.sync_copy(rv, agg_sh.at[dst_v.at[jj]], add=True)

        for wi in range(NUM_WINDOWS):
            win_base = (wid * CHUNKS_PER_TILE + wi * IDX_WIN) * CHUNK
            pltpu.sync_copy(src_hbm.at[wid].at[wi], src_v)
            pltpu.sync_copy(dst_hbm.at[wid].at[wi], dst_v)

            for b in range(2):
                fetch(win_base, b, b)

            @pl.loop(0, IDX_WIN - 2, step=2)
            def _(j):
                for b in range(2):
                    consume(win_base, j + b, b)
                    fetch(win_base, j + b + 2, b)

            for b in range(2):
                consume(win_base, IDX_WIN - 2 + b, b)

        plsc.subcore_barrier()
        pltpu.sync_copy(agg_sh.at[pl.ds(row0, ROWS_PER_TILE)],
                        agg_out.at[c].at[pl.ds(row0, ROWS_PER_TILE)])

    return sc_kernel(nf, w_edge, src_r, dst_r, z128)


def _combine_body(aggp_ref, cntp_ref, nsc_ref, na_ref, w2_ref, wa_ref, out_ref):
    agg = aggp_ref[0] + aggp_ref[1]
    cnt = (cntp_ref[0] + cntp_ref[1])[:, 0:1]
    nf2 = agg / jnp.maximum(cnt, 1.0)
    na = na_ref[...]
    wa = wa_ref[...]
    conv = jnp.dot(nf2, w2_ref[0], preferred_element_type=F32) * na[:, 0:1]
    alpha = jnp.sum(nf2 * wa[0:1, :], axis=1, keepdims=True) * na[:, 0:1]
    for j in range(1, D_ATTR):
        conv += jnp.dot(nf2, w2_ref[j], preferred_element_type=F32) * na[:, j : j + 1]
        alpha += jnp.sum(nf2 * wa[j : j + 1, :], axis=1, keepdims=True) * na[:, j : j + 1]
    out_ref[...] = nsc_ref[...] + (alpha * INV_SQRT_FANIN) * (conv * INV_SQRT_FANIN)


def _combine_call(aggp, cntp, nsc, node_attr, w2_t, wa2):
    blk = 1000
    grid = N // blk
    return pl.pallas_call(
        _combine_body,
        grid=(grid,),
        in_specs=[
            pl.BlockSpec((2, blk, D_IN), lambda i: (0, i, 0)),
            pl.BlockSpec((2, blk, D_IN), lambda i: (0, i, 0)),
            pl.BlockSpec((blk, D_IN), lambda i: (i, 0)),
            pl.BlockSpec((blk, D_ATTR), lambda i: (i, 0)),
            pl.BlockSpec((D_ATTR, D_IN, D_IN), lambda i: (0, 0, 0)),
            pl.BlockSpec((D_ATTR, D_IN), lambda i: (0, 0)),
        ],
        out_specs=pl.BlockSpec((blk, D_IN), lambda i: (i, 0)),
        out_shape=jax.ShapeDtypeStruct((N, D_IN), F32),
    )(aggp, cntp, nsc, node_attr, w2_t, wa2)


def kernel(x, node_attr, edge_index, edge_attr_sh, edge_scalars,
           W_sc, W_lin1, fc_w1, fc_w2, W_lin2, W_alpha):
    wcat = jnp.concatenate(
        [W_sc.transpose(1, 0, 2), W_lin1.transpose(1, 0, 2)], axis=-1)
    fc2_t = fc_w2.reshape(FC_HIDDEN, D_IN, D_ATTR).transpose(2, 0, 1)
    w2_t = W_lin2.transpose(1, 0, 2)
    wa2 = W_alpha[:, :, 0].transpose(1, 0)

    es_pad = jnp.pad(edge_scalars, ((0, E_PAD - E), (0, 0)))
    sh_pad = jnp.pad(edge_attr_sh, ((0, E_PAD - E), (0, 0)))
    src_r = jnp.pad(edge_index[0], (0, E_PAD - E)).reshape(
        NUM_TILES, NUM_WINDOWS, IDX_WIN, CHUNK)
    dst_r = jnp.pad(edge_index[1], (0, E_PAD - E),
                    constant_values=N).reshape(NUM_TILES, NUM_WINDOWS, IDX_WIN, CHUNK)
    z128 = jnp.zeros((N_PAD, D_IN), F32)

    cntp = _sc_cnt_call(dst_r, z128)
    nsc, nf = _nodes_call(x, node_attr, wcat)
    w_edge = _edges_call(es_pad, sh_pad, fc_w1, fc2_t)
    aggp = _sc_call(nf, w_edge, src_r, dst_r, z128)
    return _combine_call(aggp, cntp, nsc, node_attr, w2_t, wa2)
